# baseline (speedup 1.0000x reference)
import functools

import jax
import jax.numpy as jnp
from jax import lax
from jax.experimental import pallas as pl
from jax.experimental.pallas import tpu as pltpu
from jax.experimental.pallas import tpu_sc as plsc

NC = 2
NS = 16
NW = NC * NS
NBUF = 8
NGATHER = NBUF - 2


def _emb_body(n_chunks, chunk, b_per_w, table_hbm, idx_hbm, out_hbm,
              idx_v, rows_v, sem_g, sem_o):
    wid = lax.axis_index("s") * NC + lax.axis_index("c")
    base = wid * b_per_w

    def load_idx(i, j):
        pltpu.sync_copy(idx_hbm.at[pl.ds(base + i * chunk, chunk)],
                        idx_v.at[j])

    def start_gather(j):
        pltpu.make_async_copy(table_hbm.at[idx_v.at[j]], rows_v.at[j],
                              sem_g.at[j]).start()

    def wait_gather(j):
        pltpu.make_async_copy(table_hbm.at[idx_v.at[j]], rows_v.at[j],
                              sem_g.at[j]).wait()

    def start_store(i, j):
        pltpu.make_async_copy(rows_v.at[j],
                              out_hbm.at[pl.ds(base + i * chunk, chunk)],
                              sem_o.at[j]).start()

    def wait_store(i, j):
        pltpu.make_async_copy(rows_v.at[j],
                              out_hbm.at[pl.ds(base + i * chunk, chunk)],
                              sem_o.at[j]).wait()

    g_fly = NGATHER

    for i in range(g_fly):
        load_idx(i, i % NBUF)
        start_gather(i % NBUF)

    for i in range(2):
        ji = i % NBUF
        wait_gather(ji)
        start_store(i, ji)
        jn = (i + g_fly) % NBUF
        load_idx(i + g_fly, jn)
        start_gather(jn)

    def outer(g, carry):
        i0 = 2 + g * NBUF
        for j in range(NBUF):
            i = i0 + j
            ji = (2 + j) % NBUF
            wait_gather(ji)
            start_store(i, ji)
            js = j % NBUF
            wait_store(i - 2, js)
            load_idx(i + g_fly, js)
            start_gather(js)
        return carry

    lax.fori_loop(0, (n_chunks - g_fly - 2) // NBUF, outer, 0)

    for k in range(g_fly):
        i = n_chunks - g_fly + k
        ji = i % NBUF
        wait_gather(ji)
        start_store(i, ji)
    for k in range(NBUF):
        i = n_chunks - NBUF + k
        wait_store(i, i % NBUF)


def kernel(vocab_id_list, table):
    batch, hist = vocab_id_list.shape
    vocab, d = table.shape
    b = batch * hist
    idx = vocab_id_list.reshape(b).astype(jnp.int32)

    b_per_w = b // NW
    chunk = 400
    while b_per_w % chunk or (b_per_w // chunk - NGATHER - 2) % NBUF:
        chunk //= 2
    n_chunks = b_per_w // chunk

    mesh = plsc.VectorSubcoreMesh(core_axis_name="c", subcore_axis_name="s")
    out = pl.kernel(
        functools.partial(_emb_body, n_chunks, chunk, b_per_w),
        out_type=jax.ShapeDtypeStruct((b, d), jnp.float32),
        mesh=mesh,
        compiler_params=pltpu.CompilerParams(use_tc_tiling_on_sc=False),
        scratch_types=[
            pltpu.VMEM((NBUF, chunk), jnp.int32),
            pltpu.VMEM((NBUF, chunk, d), jnp.float32),
            pltpu.SemaphoreType.DMA((NBUF,)),
            pltpu.SemaphoreType.DMA((NBUF,)),
        ],
    )(table, idx)
    return out.reshape(batch, hist, d)
